# Initial kernel scaffold; baseline (speedup 1.0000x reference)
import jax
import jax.numpy as jnp
from jax.experimental import pallas as pl


def kernel(x, embed_weight, pe):
    raise NotImplementedError("write your pallas kernel here")


# speedup vs baseline: 2.7082x; 2.7082x over previous
import functools

import jax
import jax.numpy as jnp
from jax import lax
from jax.experimental import pallas as pl
from jax.experimental.pallas import tpu as pltpu
from jax.experimental.pallas import tpu_sc as plsc

B, P, V, D = 16384, 50, 39, 32
NC, NS = 2, 16
NW = NC * NS
R = B * P
RPW = R // NW
GR = 128
NJ = RPW // GR


def _prep_body(x_ref, w_ref, pe_ref, c_ref, fused_ref):
    off = lax.broadcasted_iota(jnp.int32, (B, P), 1) * V
    c_ref[...] = x_ref[...] + off
    fused_ref[...] = pe_ref[0][:, None, :] + w_ref[...][None, :, :]


_prep = pl.pallas_call(
    _prep_body,
    out_shape=(
        jax.ShapeDtypeStruct((B, P), jnp.int32),
        jax.ShapeDtypeStruct((P, V, D), jnp.float32),
    ),
)


def _gather_body(fused_hbm, c_hbm, out_hbm, idx_v, rows_v, sem):
    wid = lax.axis_index("s") * NC + lax.axis_index("c")
    pltpu.sync_copy(c_hbm.at[wid], idx_v)
    base = wid * RPW

    def body(j, carry):
        pltpu.async_copy(fused_hbm.at[idx_v.at[j]], rows_v, sem).wait()
        pltpu.sync_copy(rows_v, out_hbm.at[pl.ds(base + j * GR, GR)])
        return carry

    lax.fori_loop(0, NJ, body, 0)


_gather = functools.partial(
    pl.kernel,
    out_type=jax.ShapeDtypeStruct((R, D), jnp.float32),
    mesh=plsc.VectorSubcoreMesh(core_axis_name="c", subcore_axis_name="s"),
    scratch_types=[
        pltpu.VMEM((NJ, GR), jnp.int32),
        pltpu.VMEM((GR, D), jnp.float32),
        pltpu.SemaphoreType.DMA,
    ],
    compiler_params=pltpu.CompilerParams(use_tc_tiling_on_sc=False),
)(_gather_body)


def kernel(x, embed_weight, pe):
    x = x.astype(jnp.int32)
    c, fused = _prep(x, embed_weight, pe)
    out = _gather(fused.reshape(P * V, D), c.reshape(NW, NJ, GR))
    return out.reshape(B, P, D)


# speedup vs baseline: 2.8601x; 1.0561x over previous
import functools

import jax
import jax.numpy as jnp
from jax import lax
from jax.experimental import pallas as pl
from jax.experimental.pallas import tpu as pltpu
from jax.experimental.pallas import tpu_sc as plsc

B, P, V, D = 16384, 50, 39, 32
NC, NS = 2, 16
NW = NC * NS
R = B * P
RPW = R // NW
GR = 128
NJ = RPW // GR


def _prep_body(x_ref, w_ref, pe_ref, c_ref, fused_ref):
    off = lax.broadcasted_iota(jnp.int32, (B, P), 1) * V
    c_ref[...] = x_ref[...] + off
    fused_ref[...] = pe_ref[0][:, None, :] + w_ref[...][None, :, :]


_prep = pl.pallas_call(
    _prep_body,
    out_shape=(
        jax.ShapeDtypeStruct((B, P), jnp.int32),
        jax.ShapeDtypeStruct((P, V, D), jnp.float32),
    ),
)


KB = 8
GRP = KB * GR
NG = RPW // GRP


def _gather_body(fused_hbm, c_hbm, out_hbm, idx_v, rows_v, gsem, wsem):
    wid = lax.axis_index("s") * NC + lax.axis_index("c")
    pltpu.sync_copy(c_hbm.at[wid], idx_v)
    base = wid * RPW

    def run_group(g, drain_prev_write):
        p = g % 2
        if drain_prev_write:
            pltpu.make_async_copy(
                rows_v.at[0], out_hbm.at[pl.ds(0, GRP)], wsem
            ).wait()
        descs = [
            pltpu.async_copy(
                fused_hbm.at[idx_v.at[g * KB + b]],
                rows_v.at[p, pl.ds(b * GR, GR)],
                gsem,
            )
            for b in range(KB)
        ]
        for d in descs:
            d.wait()
        pltpu.make_async_copy(
            rows_v.at[p], out_hbm.at[pl.ds(base + g * GRP, GRP)], wsem
        ).start()

    run_group(0, False)
    run_group(1, False)

    def body(g, carry):
        run_group(g, True)
        return carry

    lax.fori_loop(2, NG, body, 0)
    for _ in range(2):
        pltpu.make_async_copy(
            rows_v.at[0], out_hbm.at[pl.ds(0, GRP)], wsem
        ).wait()


_gather = functools.partial(
    pl.kernel,
    out_type=jax.ShapeDtypeStruct((R, D), jnp.float32),
    mesh=plsc.VectorSubcoreMesh(core_axis_name="c", subcore_axis_name="s"),
    scratch_types=[
        pltpu.VMEM((NJ, GR), jnp.int32),
        pltpu.VMEM((2, GRP, D), jnp.float32),
        pltpu.SemaphoreType.DMA,
        pltpu.SemaphoreType.DMA,
    ],
    compiler_params=pltpu.CompilerParams(use_tc_tiling_on_sc=False),
)(_gather_body)


def kernel(x, embed_weight, pe):
    x = x.astype(jnp.int32)
    c, fused = _prep(x, embed_weight, pe)
    out = _gather(fused.reshape(P * V, D), c.reshape(NW, NJ, GR))
    return out.reshape(B, P, D)


# speedup vs baseline: 5.5555x; 1.9424x over previous
import functools

import jax
import jax.numpy as jnp
from jax import lax
from jax.experimental import pallas as pl
from jax.experimental.pallas import tpu as pltpu
from jax.experimental.pallas import tpu_sc as plsc

B, P, V, D = 16384, 50, 39, 32
NC, NS = 2, 16
NW = NC * NS
BW = B // NW
KB = 16
NG = BW // KB


def _prep_body(x_ref, w_ref, pe_ref, c_ref, fused_ref):
    off = lax.broadcasted_iota(jnp.int32, (B, P), 1) * V
    c_ref[...] = x_ref[...] + off
    fused_ref[...] = pe_ref[0][:, None, :] + w_ref[...][None, :, :]


_prep = pl.pallas_call(
    _prep_body,
    out_shape=(
        jax.ShapeDtypeStruct((B, P), jnp.int32),
        jax.ShapeDtypeStruct((P, V, D), jnp.float32),
    ),
)


def _gather_body(fused_hbm, c_hbm, out_hbm, idx_v, rows_v, gsem, wsem):
    wid = lax.axis_index("s") * NC + lax.axis_index("c")
    pltpu.sync_copy(c_hbm.at[wid], idx_v)
    base = wid * BW

    def run_group(g, drain_prev_write):
        p = g % 2
        if drain_prev_write:
            pltpu.make_async_copy(
                rows_v.at[0], out_hbm.at[pl.ds(0, KB)], wsem
            ).wait()
        descs = [
            pltpu.async_copy(
                fused_hbm.at[idx_v.at[g * KB + k]],
                rows_v.at[p, k],
                gsem,
            )
            for k in range(KB)
        ]
        for d in descs:
            d.wait()
        pltpu.make_async_copy(
            rows_v.at[p], out_hbm.at[pl.ds(base + g * KB, KB)], wsem
        ).start()

    run_group(0, False)
    run_group(1, False)

    def body(g, carry):
        run_group(g, True)
        return carry

    lax.fori_loop(2, NG, body, 0)
    for _ in range(2):
        pltpu.make_async_copy(
            rows_v.at[0], out_hbm.at[pl.ds(0, KB)], wsem
        ).wait()


_gather = functools.partial(
    pl.kernel,
    out_type=jax.ShapeDtypeStruct((B, P, D), jnp.float32),
    mesh=plsc.VectorSubcoreMesh(core_axis_name="c", subcore_axis_name="s"),
    scratch_types=[
        pltpu.VMEM((BW, P), jnp.int32),
        pltpu.VMEM((2, KB, P, D), jnp.float32),
        pltpu.SemaphoreType.DMA,
        pltpu.SemaphoreType.DMA,
    ],
    compiler_params=pltpu.CompilerParams(use_tc_tiling_on_sc=False),
)(_gather_body)


def kernel(x, embed_weight, pe):
    x = x.astype(jnp.int32)
    c, fused = _prep(x, embed_weight, pe)
    return _gather(fused.reshape(P * V, D), c.reshape(NW, BW, P))


# speedup vs baseline: 5.5580x; 1.0005x over previous
import functools

import jax
import jax.numpy as jnp
from jax import lax
from jax.experimental import pallas as pl
from jax.experimental.pallas import tpu as pltpu
from jax.experimental.pallas import tpu_sc as plsc

B, P, V, D = 16384, 50, 39, 32
NC, NS = 2, 16
NW = NC * NS
BW = B // NW
KB = 16
NG = BW // KB
NB = 3


def _prep_body(x_ref, w_ref, pe_ref, c_ref, fused_ref):
    off = lax.broadcasted_iota(jnp.int32, (B, P), 1) * V
    c_ref[...] = x_ref[...] + off
    fused_ref[...] = pe_ref[0][:, None, :] + w_ref[...][None, :, :]


_prep = pl.pallas_call(
    _prep_body,
    out_shape=(
        jax.ShapeDtypeStruct((B, P), jnp.int32),
        jax.ShapeDtypeStruct((P, V, D), jnp.float32),
    ),
)


def _gather_body(fused_hbm, c_hbm, out_hbm, idx_v, rows_v, gsem, wsem):
    wid = lax.axis_index("s") * NC + lax.axis_index("c")
    pltpu.sync_copy(c_hbm.at[wid], idx_v)
    base = wid * BW

    def run_group(g, drain_prev_write):
        p = g % NB
        if drain_prev_write:
            pltpu.make_async_copy(
                rows_v.at[0], out_hbm.at[pl.ds(0, KB)], wsem
            ).wait()
        descs = [
            pltpu.async_copy(
                fused_hbm.at[idx_v.at[g * KB + k]],
                rows_v.at[p, k],
                gsem,
            )
            for k in range(KB)
        ]
        for d in descs:
            d.wait()
        pltpu.make_async_copy(
            rows_v.at[p], out_hbm.at[pl.ds(base + g * KB, KB)], wsem
        ).start()

    for g0 in range(NB):
        run_group(g0, False)

    def body(g, carry):
        run_group(g, True)
        return carry

    lax.fori_loop(NB, NG, body, 0)
    for _ in range(NB):
        pltpu.make_async_copy(
            rows_v.at[0], out_hbm.at[pl.ds(0, KB)], wsem
        ).wait()


_gather = functools.partial(
    pl.kernel,
    out_type=jax.ShapeDtypeStruct((B, P, D), jnp.float32),
    mesh=plsc.VectorSubcoreMesh(core_axis_name="c", subcore_axis_name="s"),
    scratch_types=[
        pltpu.VMEM((BW, P), jnp.int32),
        pltpu.VMEM((NB, KB, P, D), jnp.float32),
        pltpu.SemaphoreType.DMA,
        pltpu.SemaphoreType.DMA,
    ],
    compiler_params=pltpu.CompilerParams(use_tc_tiling_on_sc=False),
)(_gather_body)


def kernel(x, embed_weight, pe):
    x = x.astype(jnp.int32)
    c, fused = _prep(x, embed_weight, pe)
    return _gather(fused.reshape(P * V, D), c.reshape(NW, BW, P))
